# baseline (speedup 1.0000x reference)
import functools

import jax
import jax.numpy as jnp
from jax import lax
from jax.experimental import pallas as pl
from jax.experimental.pallas import tpu as pltpu
from jax.experimental.pallas import tpu_sc as plsc

EPS = 1e-08
CE_W = 0.6
IOU_W = 0.4
NUM_CLASSES = 21

NB = 512
NC = 2
NS = 16
NW = NC * NS
B_ = 4
H_ = 512
W_ = 512
HW_ = H_ * W_
WPS = NW // B_
ROWS_W = H_ // WPS
RPC = 8
CW = 256
NCHUNK = (ROWS_W // RPC) * (W_ // CW)
VPC = RPC * CW // 16
TAB = NUM_CLASSES * NB

_NBADJ = NB * (1.0 - 1e-6)


def _sc_body(probs_hbm, target_hbm, k_out, a1_out, pbuf, tbuf, ktab, a1tab,
             psem0, psem1, tsem0, tsem1):
    wid = lax.axis_index("s") * NC + lax.axis_index("c")
    s = wid // WPS
    row0 = (wid % WPS) * ROWS_W

    def zero_body(i, _):
        z = jnp.zeros((16,), jnp.float32)
        ktab[pl.ds(i * 16, 16)] = z
        a1tab[pl.ds(i * 16, 16)] = z
        return 0
    lax.fori_loop(0, TAB // 16, zero_body, 0)

    ones16 = jnp.ones((16,), jnp.float32)
    iota16 = lax.iota(jnp.int32, 16)
    zeros16i = jnp.zeros((16,), jnp.int32)
    psems = [psem0, psem1]
    tsems = [tsem0, tsem1]

    def start_fetch(g, slot):
        r0 = row0 + lax.shift_right_logical(g, 1) * RPC
        c0 = (g & 1) * CW
        pltpu.async_copy(target_hbm.at[s, pl.ds(r0, RPC), pl.ds(c0, CW)],
                         tbuf.at[slot], tsems[slot])
        pltpu.async_copy(probs_hbm.at[s, :, pl.ds(r0, RPC), pl.ds(c0, CW)],
                         pbuf.at[slot], psems[slot])

    def wait_fetch(slot):
        pltpu.make_async_copy(target_hbm.at[s, pl.ds(0, RPC), pl.ds(0, CW)],
                              tbuf.at[slot], tsems[slot]).wait()
        pltpu.make_async_copy(probs_hbm.at[s, :, pl.ds(0, RPC), pl.ds(0, CW)],
                              pbuf.at[slot], psems[slot]).wait()

    def compute_chunk(slot):
        def vec_body(i):
            r = lax.shift_right_logical(i, 4)
            cb = (i & 15) * 16
            t = tbuf[slot, r, pl.ds(cb, 16)]
            for c in range(NUM_CLASSES):
                p = pbuf[slot, c, r, pl.ds(cb, 16)]
                bi = (p * _NBADJ).astype(jnp.int32) + (c * NB)
                plsc.addupdate_scatter(a1tab, [bi], p)
            ptraw = plsc.load_gather(
                pbuf, [zeros16i + slot, t, zeros16i + r, cb + iota16])
            tnb = t * NB
            bip = (ptraw * _NBADJ).astype(jnp.int32) + tnb
            plsc.addupdate_scatter(a1tab, [bip], -ptraw)
            efg = 1.0 - ptraw
            bifg = (efg * _NBADJ).astype(jnp.int32) + tnb
            plsc.addupdate_scatter(a1tab, [bifg], efg)
            plsc.addupdate_scatter(ktab, [bifg], ones16)

        plsc.parallel_loop(0, VPC, 1, unroll=4)(vec_body)

    start_fetch(0, 0)

    def pair_body(g2, carry):
        g = g2 * 2
        start_fetch(g + 1, 1)
        wait_fetch(0)
        compute_chunk(0)

        @pl.when(g + 2 < NCHUNK)
        def _():
            start_fetch(g + 2, 0)

        wait_fetch(1)
        compute_chunk(1)
        return carry

    lax.fori_loop(0, NCHUNK // 2, pair_body, 0)

    pltpu.sync_copy(ktab, k_out.at[wid])
    pltpu.sync_copy(a1tab, a1_out.at[wid])


_sc_pass = functools.partial(
    pl.kernel,
    mesh=plsc.VectorSubcoreMesh(core_axis_name="c", subcore_axis_name="s"),
    compiler_params=pltpu.CompilerParams(needs_layout_passes=False),
    out_type=(
        jax.ShapeDtypeStruct((NW, TAB), jnp.float32),
        jax.ShapeDtypeStruct((NW, TAB), jnp.float32),
    ),
    scratch_types=[
        pltpu.VMEM((2, NUM_CLASSES, RPC, CW), jnp.float32),
        pltpu.VMEM((2, RPC, CW), jnp.int32),
        pltpu.VMEM((TAB,), jnp.float32),
        pltpu.VMEM((TAB,), jnp.float32),
        pltpu.SemaphoreType.DMA,
        pltpu.SemaphoreType.DMA,
        pltpu.SemaphoreType.DMA,
        pltpu.SemaphoreType.DMA,
    ],
)(_sc_body)


CE_ROWS = 64
CE_STEPS = B_ * H_ // CE_ROWS


def _ce_body(p_ref, t_ref, w_ref, out_ref):
    step = pl.program_id(0) * pl.num_programs(1) + pl.program_id(1)
    t = t_ref[0]
    sump = jnp.zeros((CE_ROWS, W_), jnp.float32)
    pt = jnp.zeros((CE_ROWS, W_), jnp.float32)
    wt = jnp.zeros((CE_ROWS, W_), jnp.float32)
    for c in range(NUM_CLASSES):
        pc = jnp.maximum(p_ref[0, c], EPS)
        sump = sump + pc
        fg = t == c
        pt = jnp.where(fg, pc, pt)
        wt = wt + jnp.where(fg, w_ref[0, c], 0.0)
    num = jnp.sum(wt * (jnp.log(pt) - jnp.log(sump)))
    den = jnp.sum(wt)

    @pl.when(step == 0)
    def _():
        out_ref[0, 0] = 0.0
        out_ref[0, 1] = 0.0

    out_ref[0, 0] += num
    out_ref[0, 1] += den


def _combine_body(k_ref, a1_ref, ce_ref, out_ref):
    K = jnp.sum(k_ref[...], axis=1)
    A1 = jnp.sum(a1_ref[...], axis=1)

    Kf = K.reshape(B_ * NUM_CLASSES, NB)
    r = lax.broadcasted_iota(jnp.int32, (NB, NB), 0)
    cidx = lax.broadcasted_iota(jnp.int32, (NB, NB), 1)
    upper = (r > cidx).astype(jnp.float32)
    C_gt = jnp.dot(Kf, upper, preferred_element_type=jnp.float32)
    C_gt = C_gt.reshape(B_, NUM_CLASSES, NB)

    b = lax.broadcasted_iota(jnp.int32, (B_, NUM_CLASSES, NB), 2).astype(jnp.float32)
    mid = (b + 0.5) / NB
    w = 1.0 / NB
    loss_sum = jnp.sum(
        A1 * (C_gt + 0.5 * K) + K * (mid * 0.5 + w / 12.0),
        axis=2)

    fgcnt = jnp.sum(K, axis=2)
    denom = jnp.maximum(fgcnt, 1.0)
    loss_c = loss_sum / (denom * HW_)
    present = (jnp.sum(fgcnt, axis=0) > 0.0).astype(jnp.float32)
    total = jnp.sum(present[None, :] * loss_c)
    count = jnp.sum(present) * B_
    loss_iou = jnp.where(count > 0.0,
                         total / jnp.maximum(count, 1.0),
                         jnp.float32(0.0))

    loss_ce = -ce_ref[0, 0] / ce_ref[0, 1]

    out_ref[0, 0] = CE_W * loss_ce + IOU_W * loss_iou


def kernel(probs, target, ce_weight):
    probs = probs.astype(jnp.float32)
    target = target.astype(jnp.int32)
    cew2 = ce_weight.astype(jnp.float32).reshape(1, NUM_CLASSES)

    K, A1 = _sc_pass(probs, target)

    ce = pl.pallas_call(
        _ce_body,
        grid=(B_, H_ // CE_ROWS),
        in_specs=[
            pl.BlockSpec((1, NUM_CLASSES, CE_ROWS, W_),
                         lambda i, j: (i, 0, j, 0)),
            pl.BlockSpec((1, CE_ROWS, W_), lambda i, j: (i, j, 0)),
            pl.BlockSpec(memory_space=pltpu.SMEM),
        ],
        out_specs=pl.BlockSpec(memory_space=pltpu.SMEM),
        out_shape=jax.ShapeDtypeStruct((1, 2), jnp.float32),
    )(probs, target, cew2)

    K4 = K.reshape(B_, WPS, NUM_CLASSES, NB)
    A14 = A1.reshape(B_, WPS, NUM_CLASSES, NB)

    out = pl.pallas_call(
        _combine_body,
        in_specs=[
            pl.BlockSpec(memory_space=pltpu.VMEM),
            pl.BlockSpec(memory_space=pltpu.VMEM),
            pl.BlockSpec(memory_space=pltpu.SMEM),
        ],
        out_specs=pl.BlockSpec(memory_space=pltpu.SMEM),
        out_shape=jax.ShapeDtypeStruct((1, 1), jnp.float32),
    )(K4, A14, ce)
    return out[0, 0]


# speedup vs baseline: 1.0504x; 1.0504x over previous
import functools

import jax
import jax.numpy as jnp
from jax import lax
from jax.experimental import pallas as pl
from jax.experimental.pallas import tpu as pltpu
from jax.experimental.pallas import tpu_sc as plsc

EPS = 1e-08
CE_W = 0.6
IOU_W = 0.4
NUM_CLASSES = 21

NB = 512
NC = 2
NS = 16
NW = NC * NS
B_ = 4
H_ = 512
W_ = 512
HW_ = H_ * W_
WPS = NW // B_
ROWS_W = H_ // WPS
RPC = 8
CW = 256
NCHUNK = (ROWS_W // RPC) * (W_ // CW)
VPC = RPC * CW // 16
TAB = NUM_CLASSES * NB

_NBADJ = NB * (1.0 - 1e-6)


def _sc_body(probs_hbm, target_hbm, k_out, a1_out, pbuf, tbuf, ktab, a1tab,
             psem0, psem1, tsem0, tsem1):
    wid = lax.axis_index("s") * NC + lax.axis_index("c")
    s = wid // WPS
    row0 = (wid % WPS) * ROWS_W

    def zero_body(i, _):
        z = jnp.zeros((16,), jnp.float32)
        ktab[pl.ds(i * 16, 16)] = z
        a1tab[pl.ds(i * 16, 16)] = z
        return 0
    lax.fori_loop(0, TAB // 16, zero_body, 0)

    ones16 = jnp.ones((16,), jnp.float32)
    iota16 = lax.iota(jnp.int32, 16)
    zeros16i = jnp.zeros((16,), jnp.int32)
    psems = [psem0, psem1]
    tsems = [tsem0, tsem1]

    def start_fetch(g, slot):
        r0 = row0 + lax.shift_right_logical(g, 1) * RPC
        c0 = (g & 1) * CW
        pltpu.async_copy(target_hbm.at[s, pl.ds(r0, RPC), pl.ds(c0, CW)],
                         tbuf.at[slot], tsems[slot])
        pltpu.async_copy(probs_hbm.at[s, :, pl.ds(r0, RPC), pl.ds(c0, CW)],
                         pbuf.at[slot], psems[slot])

    def wait_fetch(slot):
        pltpu.make_async_copy(target_hbm.at[s, pl.ds(0, RPC), pl.ds(0, CW)],
                              tbuf.at[slot], tsems[slot]).wait()
        pltpu.make_async_copy(probs_hbm.at[s, :, pl.ds(0, RPC), pl.ds(0, CW)],
                              pbuf.at[slot], psems[slot]).wait()

    def compute_chunk(slot):
        def vec_body(i):
            r = lax.shift_right_logical(i, 4)
            cb = (i & 15) * 16
            t = tbuf[slot, r, pl.ds(cb, 16)]
            for c in range(NUM_CLASSES):
                p = pbuf[slot, c, r, pl.ds(cb, 16)]
                bi = (p * _NBADJ).astype(jnp.int32) + (c * NB)
                plsc.addupdate_scatter(a1tab, [bi], p)
            ptraw = plsc.load_gather(
                pbuf, [zeros16i + slot, t, zeros16i + r, cb + iota16])
            tnb = t * NB
            bip = (ptraw * _NBADJ).astype(jnp.int32) + tnb
            plsc.addupdate_scatter(a1tab, [bip], -ptraw)
            efg = 1.0 - ptraw
            bifg = (efg * _NBADJ).astype(jnp.int32) + tnb
            plsc.addupdate_scatter(a1tab, [bifg], efg)
            plsc.addupdate_scatter(ktab, [bifg], ones16)

        plsc.parallel_loop(0, VPC, 1, unroll=2)(vec_body)

    start_fetch(0, 0)

    def pair_body(g2, carry):
        g = g2 * 2
        start_fetch(g + 1, 1)
        wait_fetch(0)
        compute_chunk(0)

        @pl.when(g + 2 < NCHUNK)
        def _():
            start_fetch(g + 2, 0)

        wait_fetch(1)
        compute_chunk(1)
        return carry

    lax.fori_loop(0, NCHUNK // 2, pair_body, 0)

    pltpu.sync_copy(ktab, k_out.at[wid])
    pltpu.sync_copy(a1tab, a1_out.at[wid])


_sc_pass = functools.partial(
    pl.kernel,
    mesh=plsc.VectorSubcoreMesh(core_axis_name="c", subcore_axis_name="s"),
    compiler_params=pltpu.CompilerParams(needs_layout_passes=False),
    out_type=(
        jax.ShapeDtypeStruct((NW, TAB), jnp.float32),
        jax.ShapeDtypeStruct((NW, TAB), jnp.float32),
    ),
    scratch_types=[
        pltpu.VMEM((2, NUM_CLASSES, RPC, CW), jnp.float32),
        pltpu.VMEM((2, RPC, CW), jnp.int32),
        pltpu.VMEM((TAB,), jnp.float32),
        pltpu.VMEM((TAB,), jnp.float32),
        pltpu.SemaphoreType.DMA,
        pltpu.SemaphoreType.DMA,
        pltpu.SemaphoreType.DMA,
        pltpu.SemaphoreType.DMA,
    ],
)(_sc_body)


CE_ROWS = 64
CE_STEPS = B_ * H_ // CE_ROWS


def _ce_body(p_ref, t_ref, w_ref, out_ref):
    step = pl.program_id(0) * pl.num_programs(1) + pl.program_id(1)
    t = t_ref[0]
    sump = jnp.zeros((CE_ROWS, W_), jnp.float32)
    pt = jnp.zeros((CE_ROWS, W_), jnp.float32)
    wt = jnp.zeros((CE_ROWS, W_), jnp.float32)
    for c in range(NUM_CLASSES):
        pc = jnp.maximum(p_ref[0, c], EPS)
        sump = sump + pc
        fg = t == c
        pt = jnp.where(fg, pc, pt)
        wt = wt + jnp.where(fg, w_ref[0, c], 0.0)
    num = jnp.sum(wt * (jnp.log(pt) - jnp.log(sump)))
    den = jnp.sum(wt)

    @pl.when(step == 0)
    def _():
        out_ref[0, 0] = 0.0
        out_ref[0, 1] = 0.0

    out_ref[0, 0] += num
    out_ref[0, 1] += den


def _combine_body(k_ref, a1_ref, ce_ref, out_ref):
    K = jnp.sum(k_ref[...], axis=1)
    A1 = jnp.sum(a1_ref[...], axis=1)

    Kf = K.reshape(B_ * NUM_CLASSES, NB)
    r = lax.broadcasted_iota(jnp.int32, (NB, NB), 0)
    cidx = lax.broadcasted_iota(jnp.int32, (NB, NB), 1)
    upper = (r > cidx).astype(jnp.float32)
    C_gt = jnp.dot(Kf, upper, preferred_element_type=jnp.float32)
    C_gt = C_gt.reshape(B_, NUM_CLASSES, NB)

    b = lax.broadcasted_iota(jnp.int32, (B_, NUM_CLASSES, NB), 2).astype(jnp.float32)
    mid = (b + 0.5) / NB
    w = 1.0 / NB
    loss_sum = jnp.sum(
        A1 * (C_gt + 0.5 * K) + K * (mid * 0.5 + w / 12.0),
        axis=2)

    fgcnt = jnp.sum(K, axis=2)
    denom = jnp.maximum(fgcnt, 1.0)
    loss_c = loss_sum / (denom * HW_)
    present = (jnp.sum(fgcnt, axis=0) > 0.0).astype(jnp.float32)
    total = jnp.sum(present[None, :] * loss_c)
    count = jnp.sum(present) * B_
    loss_iou = jnp.where(count > 0.0,
                         total / jnp.maximum(count, 1.0),
                         jnp.float32(0.0))

    loss_ce = -ce_ref[0, 0] / ce_ref[0, 1]

    out_ref[0, 0] = CE_W * loss_ce + IOU_W * loss_iou


def kernel(probs, target, ce_weight):
    probs = probs.astype(jnp.float32)
    target = target.astype(jnp.int32)
    cew2 = ce_weight.astype(jnp.float32).reshape(1, NUM_CLASSES)

    K, A1 = _sc_pass(probs, target)

    ce = pl.pallas_call(
        _ce_body,
        grid=(B_, H_ // CE_ROWS),
        in_specs=[
            pl.BlockSpec((1, NUM_CLASSES, CE_ROWS, W_),
                         lambda i, j: (i, 0, j, 0)),
            pl.BlockSpec((1, CE_ROWS, W_), lambda i, j: (i, j, 0)),
            pl.BlockSpec(memory_space=pltpu.SMEM),
        ],
        out_specs=pl.BlockSpec(memory_space=pltpu.SMEM),
        out_shape=jax.ShapeDtypeStruct((1, 2), jnp.float32),
    )(probs, target, cew2)

    K4 = K.reshape(B_, WPS, NUM_CLASSES, NB)
    A14 = A1.reshape(B_, WPS, NUM_CLASSES, NB)

    out = pl.pallas_call(
        _combine_body,
        in_specs=[
            pl.BlockSpec(memory_space=pltpu.VMEM),
            pl.BlockSpec(memory_space=pltpu.VMEM),
            pl.BlockSpec(memory_space=pltpu.SMEM),
        ],
        out_specs=pl.BlockSpec(memory_space=pltpu.SMEM),
        out_shape=jax.ShapeDtypeStruct((1, 1), jnp.float32),
    )(K4, A14, ce)
    return out[0, 0]
